# baseline (speedup 1.0000x reference)
import functools

import jax
import jax.numpy as jnp
from jax import lax
from jax.experimental import pallas as pl
from jax.experimental.pallas import tpu as pltpu
from jax.experimental.pallas import tpu_sc as plsc

_INFO = plsc.get_sparse_core_info()
_NC, _NS = _INFO.num_cores, _INFO.num_subcores
_NW = _NC * _NS


def kernel(new_k, new_v):
    shape = new_k.shape
    k2 = new_k.reshape(-1, shape[-1])
    v2 = new_v.reshape(-1, shape[-1])
    rows, cols = k2.shape
    rows_per = rows // _NW

    mesh = plsc.VectorSubcoreMesh(core_axis_name="c", subcore_axis_name="s")

    @functools.partial(
        pl.kernel,
        mesh=mesh,
        out_type=[
            jax.ShapeDtypeStruct(k2.shape, k2.dtype),
            jax.ShapeDtypeStruct(v2.shape, v2.dtype),
        ],
        scratch_types=[pltpu.SemaphoreType.DMA],
    )
    def _sc_copy(k_hbm, v_hbm, k_out, v_out, sem):
        wid = lax.axis_index("s") * _NC + lax.axis_index("c")
        base = wid * rows_per
        sl = pl.ds(base, rows_per)
        ck = pltpu.make_async_copy(k_hbm.at[sl], k_out.at[sl], sem)
        cv = pltpu.make_async_copy(v_hbm.at[sl], v_out.at[sl], sem)
        ck.start()
        cv.start()
        ck.wait()
        cv.wait()

    out = _sc_copy(k2, v2)
    return (out[0].reshape(shape), out[1].reshape(shape))
